# Initial kernel scaffold; baseline (speedup 1.0000x reference)
import jax
import jax.numpy as jnp
from jax.experimental import pallas as pl


def kernel(input_ids, tok_embed, pos_embed, ln1_s, ln1_b, Wqkv, bqkv, Wo, bo, ln2_s, ln2_b, W1, b1, W2, b2, Wq_mem, bq_mem, Wr_mem, br_mem, out_s, out_b, K0, V0, S0, K1, V1, S1, K2, V2, S2):
    raise NotImplementedError("write your pallas kernel here")


# speedup vs baseline: 8.9454x; 8.9454x over previous
import functools
import math

import jax
import jax.numpy as jnp
from jax.experimental import pallas as pl
from jax.experimental.pallas import tpu as pltpu

VOCAB = 32000
D = 512
NL = 4
NH = 8
DH = 64
T = 1024
B = 2
R = B * T
RB = 256
NR = R // RB
FF = 2048
TOPK = 4

VBLK_E = 1280
NV_E = VOCAB // VBLK_E
VBLK_L = 3200
NV_L = VOCAB // VBLK_L
SBLK = 2048

_NEG = float("-inf")
_BIGI = 2**30


def _lnf(x, s, b):
    m = jnp.mean(x, axis=1, keepdims=True)
    v = jnp.mean((x - m) ** 2, axis=1, keepdims=True)
    return (x - m) / jnp.sqrt(v + 1e-5) * s + b


def _embed_body(ids_ref, pos_ref, emb_ref, out_ref):
    v = pl.program_id(0)

    @pl.when(v == 0)
    def _():
        p = pos_ref[...]
        out_ref[0:T, :] = p
        out_ref[T:R, :] = p

    ids = ids_ref[...]
    iota = jax.lax.broadcasted_iota(jnp.int32, (R, VBLK_E), 1) + v * VBLK_E
    oh = jnp.where(iota == ids, jnp.float32(1.0), jnp.float32(0.0))
    out_ref[...] += jnp.dot(oh, emb_ref[...], preferred_element_type=jnp.float32)


def _embed(ids, pos, emb):
    return pl.pallas_call(
        _embed_body,
        grid=(NV_E,),
        in_specs=[
            pl.BlockSpec((R, 1), lambda v: (0, 0)),
            pl.BlockSpec((T, D), lambda v: (0, 0)),
            pl.BlockSpec((VBLK_E, D), lambda v: (v, 0)),
        ],
        out_specs=pl.BlockSpec((R, D), lambda v: (0, 0)),
        out_shape=jax.ShapeDtypeStruct((R, D), jnp.float32),
    )(ids, pos, emb)


def _qkv_body(x_ref, s_ref, b_ref, w_ref, bias_ref, out_ref):
    h = _lnf(x_ref[...], s_ref[...], b_ref[...])
    out_ref[...] = jnp.dot(h, w_ref[...], preferred_element_type=jnp.float32) + bias_ref[...]


def _qkv(x, s, b, w, bias):
    return pl.pallas_call(
        _qkv_body,
        grid=(NR,),
        in_specs=[
            pl.BlockSpec((RB, D), lambda r: (r, 0)),
            pl.BlockSpec((1, D), lambda r: (0, 0)),
            pl.BlockSpec((1, D), lambda r: (0, 0)),
            pl.BlockSpec((D, 3 * D), lambda r: (0, 0)),
            pl.BlockSpec((1, 3 * D), lambda r: (0, 0)),
        ],
        out_specs=pl.BlockSpec((RB, 3 * D), lambda r: (r, 0)),
        out_shape=jax.ShapeDtypeStruct((R, 3 * D), jnp.float32),
    )(x, s, b, w, bias)


def _attn_body(qkv_ref, o_ref):
    ri = jax.lax.broadcasted_iota(jnp.int32, (T, T), 0)
    ci = jax.lax.broadcasted_iota(jnp.int32, (T, T), 1)
    causal = ri >= ci
    for h in range(NH):
        q = qkv_ref[:, h * DH:(h + 1) * DH]
        k = qkv_ref[:, D + h * DH:D + (h + 1) * DH]
        v = qkv_ref[:, 2 * D + h * DH:2 * D + (h + 1) * DH]
        s = jax.lax.dot_general(q, k, (((1,), (1,)), ((), ())),
                                preferred_element_type=jnp.float32)
        s = s * (1.0 / math.sqrt(DH))
        s = jnp.where(causal, s, jnp.float32(-1e9))
        m = jnp.max(s, axis=1, keepdims=True)
        e = jnp.exp(s - m)
        p = e / jnp.sum(e, axis=1, keepdims=True)
        o_ref[:, h * DH:(h + 1) * DH] = jnp.dot(
            p, v, preferred_element_type=jnp.float32)


def _attn(qkv):
    return pl.pallas_call(
        _attn_body,
        grid=(B,),
        in_specs=[pl.BlockSpec((T, 3 * D), lambda b: (b, 0))],
        out_specs=pl.BlockSpec((T, D), lambda b: (b, 0)),
        out_shape=jax.ShapeDtypeStruct((R, D), jnp.float32),
    )(qkv)


def _post_body(x_ref, o_ref, Wo_ref, bo_ref, s2_ref, b2_ref, W1_ref, b1_ref,
               W2_ref, b2m_ref, out_ref):
    x = x_ref[...] + jnp.dot(o_ref[...], Wo_ref[...],
                             preferred_element_type=jnp.float32) + bo_ref[...]
    h2 = _lnf(x, s2_ref[...], b2_ref[...])
    ff = jax.nn.gelu(jnp.dot(h2, W1_ref[...],
                             preferred_element_type=jnp.float32) + b1_ref[...])
    out_ref[...] = x + jnp.dot(ff, W2_ref[...],
                               preferred_element_type=jnp.float32) + b2m_ref[...]


def _post(x, o, Wo_l, bo_l, s2, b2, W1_l, b1_l, W2_l, b2m):
    return pl.pallas_call(
        _post_body,
        grid=(NR,),
        in_specs=[
            pl.BlockSpec((RB, D), lambda r: (r, 0)),
            pl.BlockSpec((RB, D), lambda r: (r, 0)),
            pl.BlockSpec((D, D), lambda r: (0, 0)),
            pl.BlockSpec((1, D), lambda r: (0, 0)),
            pl.BlockSpec((1, D), lambda r: (0, 0)),
            pl.BlockSpec((1, D), lambda r: (0, 0)),
            pl.BlockSpec((D, FF), lambda r: (0, 0)),
            pl.BlockSpec((1, FF), lambda r: (0, 0)),
            pl.BlockSpec((FF, D), lambda r: (0, 0)),
            pl.BlockSpec((1, D), lambda r: (0, 0)),
        ],
        out_specs=pl.BlockSpec((RB, D), lambda r: (r, 0)),
        out_shape=jax.ShapeDtypeStruct((R, D), jnp.float32),
    )(x, o, Wo_l, bo_l, s2, b2, W1_l, b1_l, W2_l, b2m)


def _rowmm_body(x_ref, w_ref, b_ref, out_ref):
    out_ref[...] = jnp.dot(x_ref[...], w_ref[...],
                           preferred_element_type=jnp.float32) + b_ref[...]


def _rowmm(x, w, b):
    return pl.pallas_call(
        _rowmm_body,
        grid=(NR,),
        in_specs=[
            pl.BlockSpec((RB, D), lambda r: (r, 0)),
            pl.BlockSpec((D, D), lambda r: (0, 0)),
            pl.BlockSpec((1, D), lambda r: (0, 0)),
        ],
        out_specs=pl.BlockSpec((RB, D), lambda r: (r, 0)),
        out_shape=jax.ShapeDtypeStruct((R, D), jnp.float32),
    )(x, w, b)


def _topk_body(qm_ref, K_ref, Sb_ref, tv_ref, ti_ref, sv, si, *, ns):
    s_idx = pl.program_id(0)
    r_idx = pl.program_id(1)
    sc = jax.lax.dot_general(qm_ref[...], K_ref[...], (((1,), (1,)), ((), ())),
                             preferred_element_type=jnp.float32)
    sc = sc * (1.0 / math.sqrt(D)) + Sb_ref[...]
    iota = jax.lax.broadcasted_iota(jnp.int32, (RB, SBLK), 1) + s_idx * SBLK
    bvs, bis = [], []
    for _ in range(TOPK):
        m = jnp.max(sc, axis=1, keepdims=True)
        mi = jnp.min(jnp.where(sc == m, iota, _BIGI), axis=1, keepdims=True)
        bvs.append(m)
        bis.append(mi)
        sc = jnp.where(iota == mi, _NEG, sc)
    bv = jnp.concatenate(bvs, axis=1)
    bi = jnp.concatenate(bis, axis=1)

    rsl = pl.ds(r_idx * RB, RB)

    @pl.when(s_idx == 0)
    def _():
        sv[rsl, :] = jnp.full((RB, TOPK), _NEG, jnp.float32)
        si[rsl, :] = jnp.zeros((RB, TOPK), jnp.int32)

    av = jnp.concatenate([sv[rsl, :], bv], axis=1)
    ai = jnp.concatenate([si[rsl, :], bi], axis=1)
    nvs, nis = [], []
    for _ in range(TOPK):
        m = jnp.max(av, axis=1, keepdims=True)
        mi = jnp.min(jnp.where(av == m, ai, _BIGI), axis=1, keepdims=True)
        nvs.append(m)
        nis.append(mi)
        av = jnp.where(ai == mi, _NEG, av)
    sv[rsl, :] = jnp.concatenate(nvs, axis=1)
    si[rsl, :] = jnp.concatenate(nis, axis=1)

    @pl.when(s_idx == ns - 1)
    def _():
        tv_ref[...] = sv[rsl, :]
        ti_ref[...] = si[rsl, :]


def _topk(qm, K, Sb):
    S = K.shape[0]
    ns = S // SBLK
    return pl.pallas_call(
        functools.partial(_topk_body, ns=ns),
        grid=(ns, NR),
        in_specs=[
            pl.BlockSpec((RB, D), lambda s, r: (r, 0)),
            pl.BlockSpec((SBLK, D), lambda s, r: (s, 0)),
            pl.BlockSpec((1, SBLK), lambda s, r: (0, s)),
        ],
        out_specs=[
            pl.BlockSpec((RB, TOPK), lambda s, r: (r, 0)),
            pl.BlockSpec((RB, TOPK), lambda s, r: (r, 0)),
        ],
        out_shape=[
            jax.ShapeDtypeStruct((R, TOPK), jnp.float32),
            jax.ShapeDtypeStruct((R, TOPK), jnp.int32),
        ],
        scratch_shapes=[
            pltpu.VMEM((R, TOPK), jnp.float32),
            pltpu.VMEM((R, TOPK), jnp.int32),
        ],
    )(qm, K, Sb)


def _read_body(tv_ref, ti_ref, V_ref, out_ref, acc, *, ns):
    s_idx = pl.program_id(0)
    r_idx = pl.program_id(1)
    tv = tv_ref[...]
    m = jnp.max(tv, axis=1, keepdims=True)
    e = jnp.exp(tv - m)
    a = e / jnp.sum(e, axis=1, keepdims=True) * (1.0 / 3.0)
    iota = jax.lax.broadcasted_iota(jnp.int32, (RB, SBLK), 1) + s_idx * SBLK
    ti = ti_ref[...]
    A = jnp.zeros((RB, SBLK), jnp.float32)
    for kk in range(TOPK):
        A = A + jnp.where(iota == ti[:, kk:kk + 1], a[:, kk:kk + 1],
                          jnp.float32(0.0))
    contrib = jnp.dot(A, V_ref[...], preferred_element_type=jnp.float32)
    rsl = pl.ds(r_idx * RB, RB)

    @pl.when(s_idx == 0)
    def _():
        acc[rsl, :] = jnp.zeros((RB, D), jnp.float32)

    acc[rsl, :] += contrib

    @pl.when(s_idx == ns - 1)
    def _():
        out_ref[...] = acc[rsl, :]


def _read(tv, ti, V):
    S = V.shape[0]
    ns = S // SBLK
    return pl.pallas_call(
        functools.partial(_read_body, ns=ns),
        grid=(ns, NR),
        in_specs=[
            pl.BlockSpec((RB, TOPK), lambda s, r: (r, 0)),
            pl.BlockSpec((RB, TOPK), lambda s, r: (r, 0)),
            pl.BlockSpec((SBLK, D), lambda s, r: (s, 0)),
        ],
        out_specs=pl.BlockSpec((RB, D), lambda s, r: (r, 0)),
        out_shape=jax.ShapeDtypeStruct((R, D), jnp.float32),
        scratch_shapes=[pltpu.VMEM((R, D), jnp.float32)],
    )(tv, ti, V)


def _final_body(x_ref, r0_ref, r1_ref, r2_ref, Wr_ref, br_ref, s_ref, b_ref,
                out_ref):
    rd = r0_ref[...] + r1_ref[...] + r2_ref[...]
    x = x_ref[...] + jnp.dot(rd, Wr_ref[...],
                             preferred_element_type=jnp.float32) + br_ref[...]
    out_ref[...] = _lnf(x, s_ref[...], b_ref[...])


def _final(x, r0, r1, r2, Wr, br, s, b):
    return pl.pallas_call(
        _final_body,
        grid=(NR,),
        in_specs=[
            pl.BlockSpec((RB, D), lambda r: (r, 0)),
            pl.BlockSpec((RB, D), lambda r: (r, 0)),
            pl.BlockSpec((RB, D), lambda r: (r, 0)),
            pl.BlockSpec((RB, D), lambda r: (r, 0)),
            pl.BlockSpec((D, D), lambda r: (0, 0)),
            pl.BlockSpec((1, D), lambda r: (0, 0)),
            pl.BlockSpec((1, D), lambda r: (0, 0)),
            pl.BlockSpec((1, D), lambda r: (0, 0)),
        ],
        out_specs=pl.BlockSpec((RB, D), lambda r: (r, 0)),
        out_shape=jax.ShapeDtypeStruct((R, D), jnp.float32),
    )(x, r0, r1, r2, Wr, br, s, b)


def _logits_body(x_ref, emb_ref, out_ref):
    out_ref[...] = jax.lax.dot_general(
        x_ref[...], emb_ref[...], (((1,), (1,)), ((), ())),
        preferred_element_type=jnp.float32)


def _logits(x, emb):
    return pl.pallas_call(
        _logits_body,
        grid=(NV_L, NR),
        in_specs=[
            pl.BlockSpec((RB, D), lambda v, r: (r, 0)),
            pl.BlockSpec((VBLK_L, D), lambda v, r: (v, 0)),
        ],
        out_specs=pl.BlockSpec((RB, VBLK_L), lambda v, r: (r, v)),
        out_shape=jax.ShapeDtypeStruct((R, VOCAB), jnp.float32),
    )(x, emb)


def kernel(input_ids, tok_embed, pos_embed, ln1_s, ln1_b, Wqkv, bqkv, Wo, bo,
           ln2_s, ln2_b, W1, b1, W2, b2, Wq_mem, bq_mem, Wr_mem, br_mem,
           out_s, out_b, K0, V0, S0, K1, V1, S1, K2, V2, S2):
    ids = input_ids.reshape(R, 1).astype(jnp.int32)
    x = _embed(ids, pos_embed[:T], tok_embed)
    for l in range(NL):
        qkv = _qkv(x, ln1_s[l].reshape(1, D), ln1_b[l].reshape(1, D),
                   Wqkv[l], bqkv[l].reshape(1, 3 * D))
        o = _attn(qkv)
        x = _post(x, o, Wo[l], bo[l].reshape(1, D), ln2_s[l].reshape(1, D),
                  ln2_b[l].reshape(1, D), W1[l], b1[l].reshape(1, FF),
                  W2[l], b2[l].reshape(1, D))
    qm = _rowmm(x, Wq_mem, bq_mem.reshape(1, D))
    reads = []
    for Ki, Vi, Si in ((K0, V0, S0), (K1, V1, S1), (K2, V2, S2)):
        tv, ti = _topk(qm, Ki, Si.reshape(1, -1))
        reads.append(_read(tv, ti, Vi))
    xf = _final(x, reads[0], reads[1], reads[2], Wr_mem,
                br_mem.reshape(1, D), out_s.reshape(1, D), out_b.reshape(1, D))
    logits = _logits(xf, tok_embed)
    return logits.reshape(B, T, VOCAB)


# speedup vs baseline: 10.0341x; 1.1217x over previous
import functools
import math

import jax
import jax.numpy as jnp
from jax.experimental import pallas as pl
from jax.experimental.pallas import tpu as pltpu
from jax.experimental.pallas import tpu_sc as plsc

VOCAB = 32000
D = 512
NL = 4
NH = 8
DH = 64
T = 1024
B = 2
R = B * T
RB = 256
NR = R // RB
FF = 2048
TOPK = 4

VBLK_E = 1280
NV_E = VOCAB // VBLK_E
VBLK_L = 3200
NV_L = VOCAB // VBLK_L
SBLK = 2048

_NEG = float("-inf")
_BIGI = 2**30

_NC = 2
_NS = 16
_NW = _NC * _NS
_GCH = 64


def _lnf(x, s, b):
    m = jnp.mean(x, axis=1, keepdims=True)
    v = jnp.mean((x - m) ** 2, axis=1, keepdims=True)
    return (x - m) / jnp.sqrt(v + 1e-5) * s + b


def _sc_gather(idx, table):
    n = idx.shape[0]
    epw = n // _NW
    nch = epw // _GCH
    mesh = plsc.VectorSubcoreMesh(core_axis_name="c", subcore_axis_name="s",
                                  num_cores=_NC, num_subcores=_NS)

    @functools.partial(
        pl.kernel,
        mesh=mesh,
        out_type=jax.ShapeDtypeStruct((n, D), jnp.float32),
        scratch_types=[
            pltpu.VMEM((_GCH,), jnp.int32),
            pltpu.VMEM((_GCH, D), jnp.float32),
            pltpu.SemaphoreType.DMA,
        ],
    )
    def k(table_hbm, idx_hbm, out_hbm, idx_v, rows_v, sem):
        wid = jax.lax.axis_index("s") * _NC + jax.lax.axis_index("c")
        base = wid * epw
        for c in range(nch):
            off = base + c * _GCH
            pltpu.sync_copy(idx_hbm.at[pl.ds(off, _GCH)], idx_v)
            pltpu.async_copy(table_hbm.at[idx_v], rows_v, sem).wait()
            pltpu.sync_copy(rows_v, out_hbm.at[pl.ds(off, _GCH)])

    return k(table, idx)


def _addpos_body(g_ref, p_ref, out_ref):
    out_ref[...] = g_ref[...] + p_ref[...]


def _addpos(g, pos):
    npb = T // RB
    return pl.pallas_call(
        _addpos_body,
        grid=(NR,),
        in_specs=[
            pl.BlockSpec((RB, D), lambda r: (r, 0)),
            pl.BlockSpec((RB, D), lambda r: (r % npb, 0)),
        ],
        out_specs=pl.BlockSpec((RB, D), lambda r: (r, 0)),
        out_shape=jax.ShapeDtypeStruct((R, D), jnp.float32),
    )(g, pos)


def _qkv_body(x_ref, s_ref, b_ref, w_ref, bias_ref, out_ref):
    h = _lnf(x_ref[...], s_ref[...], b_ref[...])
    out_ref[...] = jnp.dot(h, w_ref[...], preferred_element_type=jnp.float32) + bias_ref[...]


def _qkv(x, s, b, w, bias):
    return pl.pallas_call(
        _qkv_body,
        grid=(NR,),
        in_specs=[
            pl.BlockSpec((RB, D), lambda r: (r, 0)),
            pl.BlockSpec((1, D), lambda r: (0, 0)),
            pl.BlockSpec((1, D), lambda r: (0, 0)),
            pl.BlockSpec((D, 3 * D), lambda r: (0, 0)),
            pl.BlockSpec((1, 3 * D), lambda r: (0, 0)),
        ],
        out_specs=pl.BlockSpec((RB, 3 * D), lambda r: (r, 0)),
        out_shape=jax.ShapeDtypeStruct((R, 3 * D), jnp.float32),
    )(x, s, b, w, bias)


def _attn_body(qkv_ref, o_ref):
    ri = jax.lax.broadcasted_iota(jnp.int32, (T, T), 0)
    ci = jax.lax.broadcasted_iota(jnp.int32, (T, T), 1)
    causal = ri >= ci
    for h in range(NH):
        q = qkv_ref[:, h * DH:(h + 1) * DH]
        k = qkv_ref[:, D + h * DH:D + (h + 1) * DH]
        v = qkv_ref[:, 2 * D + h * DH:2 * D + (h + 1) * DH]
        s = jax.lax.dot_general(q, k, (((1,), (1,)), ((), ())),
                                preferred_element_type=jnp.float32)
        s = s * (1.0 / math.sqrt(DH))
        s = jnp.where(causal, s, jnp.float32(-1e9))
        m = jnp.max(s, axis=1, keepdims=True)
        e = jnp.exp(s - m)
        p = e / jnp.sum(e, axis=1, keepdims=True)
        o_ref[:, h * DH:(h + 1) * DH] = jnp.dot(
            p, v, preferred_element_type=jnp.float32)


def _attn(qkv):
    return pl.pallas_call(
        _attn_body,
        grid=(B,),
        in_specs=[pl.BlockSpec((T, 3 * D), lambda b: (b, 0))],
        out_specs=pl.BlockSpec((T, D), lambda b: (b, 0)),
        out_shape=jax.ShapeDtypeStruct((R, D), jnp.float32),
    )(qkv)


def _post_body(x_ref, o_ref, Wo_ref, bo_ref, s2_ref, b2_ref, W1_ref, b1_ref,
               W2_ref, b2m_ref, out_ref):
    x = x_ref[...] + jnp.dot(o_ref[...], Wo_ref[...],
                             preferred_element_type=jnp.float32) + bo_ref[...]
    h2 = _lnf(x, s2_ref[...], b2_ref[...])
    ff = jax.nn.gelu(jnp.dot(h2, W1_ref[...],
                             preferred_element_type=jnp.float32) + b1_ref[...])
    out_ref[...] = x + jnp.dot(ff, W2_ref[...],
                               preferred_element_type=jnp.float32) + b2m_ref[...]


def _post(x, o, Wo_l, bo_l, s2, b2, W1_l, b1_l, W2_l, b2m):
    return pl.pallas_call(
        _post_body,
        grid=(NR,),
        in_specs=[
            pl.BlockSpec((RB, D), lambda r: (r, 0)),
            pl.BlockSpec((RB, D), lambda r: (r, 0)),
            pl.BlockSpec((D, D), lambda r: (0, 0)),
            pl.BlockSpec((1, D), lambda r: (0, 0)),
            pl.BlockSpec((1, D), lambda r: (0, 0)),
            pl.BlockSpec((1, D), lambda r: (0, 0)),
            pl.BlockSpec((D, FF), lambda r: (0, 0)),
            pl.BlockSpec((1, FF), lambda r: (0, 0)),
            pl.BlockSpec((FF, D), lambda r: (0, 0)),
            pl.BlockSpec((1, D), lambda r: (0, 0)),
        ],
        out_specs=pl.BlockSpec((RB, D), lambda r: (r, 0)),
        out_shape=jax.ShapeDtypeStruct((R, D), jnp.float32),
    )(x, o, Wo_l, bo_l, s2, b2, W1_l, b1_l, W2_l, b2m)


def _rowmm_body(x_ref, w_ref, b_ref, out_ref):
    out_ref[...] = jnp.dot(x_ref[...], w_ref[...],
                           preferred_element_type=jnp.float32) + b_ref[...]


def _rowmm(x, w, b):
    return pl.pallas_call(
        _rowmm_body,
        grid=(NR,),
        in_specs=[
            pl.BlockSpec((RB, D), lambda r: (r, 0)),
            pl.BlockSpec((D, D), lambda r: (0, 0)),
            pl.BlockSpec((1, D), lambda r: (0, 0)),
        ],
        out_specs=pl.BlockSpec((RB, D), lambda r: (r, 0)),
        out_shape=jax.ShapeDtypeStruct((R, D), jnp.float32),
    )(x, w, b)


def _topk_body(qm_ref, K_ref, Sb_ref, tv_ref, ti_ref, sv, si, *, ns):
    s_idx = pl.program_id(0)
    r_idx = pl.program_id(1)
    sc = jax.lax.dot_general(qm_ref[...], K_ref[...], (((1,), (1,)), ((), ())),
                             preferred_element_type=jnp.float32)
    sc = sc * (1.0 / math.sqrt(D)) + Sb_ref[...]
    iota = jax.lax.broadcasted_iota(jnp.int32, (RB, SBLK), 1) + s_idx * SBLK
    bvs, bis = [], []
    for _ in range(TOPK):
        m = jnp.max(sc, axis=1, keepdims=True)
        mi = jnp.min(jnp.where(sc == m, iota, _BIGI), axis=1, keepdims=True)
        bvs.append(m)
        bis.append(mi)
        sc = jnp.where(iota == mi, _NEG, sc)
    bv = jnp.concatenate(bvs, axis=1)
    bi = jnp.concatenate(bis, axis=1)

    rsl = pl.ds(r_idx * RB, RB)

    @pl.when(s_idx == 0)
    def _():
        sv[rsl, :] = jnp.full((RB, TOPK), _NEG, jnp.float32)
        si[rsl, :] = jnp.zeros((RB, TOPK), jnp.int32)

    av = jnp.concatenate([sv[rsl, :], bv], axis=1)
    ai = jnp.concatenate([si[rsl, :], bi], axis=1)
    nvs, nis = [], []
    for _ in range(TOPK):
        m = jnp.max(av, axis=1, keepdims=True)
        mi = jnp.min(jnp.where(av == m, ai, _BIGI), axis=1, keepdims=True)
        nvs.append(m)
        nis.append(mi)
        av = jnp.where(ai == mi, _NEG, av)
    sv[rsl, :] = jnp.concatenate(nvs, axis=1)
    si[rsl, :] = jnp.concatenate(nis, axis=1)

    @pl.when(s_idx == ns - 1)
    def _():
        tv_ref[...] = sv[rsl, :]
        ti_ref[...] = si[rsl, :]


def _topk(qm, K, Sb):
    S = K.shape[0]
    ns = S // SBLK
    return pl.pallas_call(
        functools.partial(_topk_body, ns=ns),
        grid=(ns, NR),
        in_specs=[
            pl.BlockSpec((RB, D), lambda s, r: (r, 0)),
            pl.BlockSpec((SBLK, D), lambda s, r: (s, 0)),
            pl.BlockSpec((1, SBLK), lambda s, r: (0, s)),
        ],
        out_specs=[
            pl.BlockSpec((RB, TOPK), lambda s, r: (r, 0)),
            pl.BlockSpec((RB, TOPK), lambda s, r: (r, 0)),
        ],
        out_shape=[
            jax.ShapeDtypeStruct((R, TOPK), jnp.float32),
            jax.ShapeDtypeStruct((R, TOPK), jnp.int32),
        ],
        scratch_shapes=[
            pltpu.VMEM((R, TOPK), jnp.float32),
            pltpu.VMEM((R, TOPK), jnp.int32),
        ],
    )(qm, K, Sb)


def _final_body(x_ref, g0_ref, g1_ref, g2_ref, tv0_ref, tv1_ref, tv2_ref,
                Wr_ref, br_ref, s_ref, b_ref, out_ref):
    rd = jnp.zeros((RB, D), jnp.float32)
    for g_ref, tv_ref in ((g0_ref, tv0_ref), (g1_ref, tv1_ref),
                          (g2_ref, tv2_ref)):
        tv = tv_ref[...]
        m = jnp.max(tv, axis=1, keepdims=True)
        e = jnp.exp(tv - m)
        a = e / jnp.sum(e, axis=1, keepdims=True) * (1.0 / 3.0)
        for kk in range(TOPK):
            rd = rd + a[:, kk:kk + 1] * g_ref[kk]
    x = x_ref[...] + jnp.dot(rd, Wr_ref[...],
                             preferred_element_type=jnp.float32) + br_ref[...]
    out_ref[...] = _lnf(x, s_ref[...], b_ref[...])


def _final(x, g0, g1, g2, tv0, tv1, tv2, Wr, br, s, b):
    gspec = pl.BlockSpec((TOPK, RB, D), lambda r: (0, r, 0))
    tspec = pl.BlockSpec((RB, TOPK), lambda r: (r, 0))
    return pl.pallas_call(
        _final_body,
        grid=(NR,),
        in_specs=[
            pl.BlockSpec((RB, D), lambda r: (r, 0)),
            gspec, gspec, gspec, tspec, tspec, tspec,
            pl.BlockSpec((D, D), lambda r: (0, 0)),
            pl.BlockSpec((1, D), lambda r: (0, 0)),
            pl.BlockSpec((1, D), lambda r: (0, 0)),
            pl.BlockSpec((1, D), lambda r: (0, 0)),
        ],
        out_specs=pl.BlockSpec((RB, D), lambda r: (r, 0)),
        out_shape=jax.ShapeDtypeStruct((R, D), jnp.float32),
    )(x, g0, g1, g2, tv0, tv1, tv2, Wr, br, s, b)


def _logits_body(x_ref, emb_ref, out_ref):
    out_ref[...] = jax.lax.dot_general(
        x_ref[...], emb_ref[...], (((1,), (1,)), ((), ())),
        preferred_element_type=jnp.float32)


def _logits(x, emb):
    return pl.pallas_call(
        _logits_body,
        grid=(NV_L, NR),
        in_specs=[
            pl.BlockSpec((RB, D), lambda v, r: (r, 0)),
            pl.BlockSpec((VBLK_L, D), lambda v, r: (v, 0)),
        ],
        out_specs=pl.BlockSpec((RB, VBLK_L), lambda v, r: (r, v)),
        out_shape=jax.ShapeDtypeStruct((R, VOCAB), jnp.float32),
    )(x, emb)


def kernel(input_ids, tok_embed, pos_embed, ln1_s, ln1_b, Wqkv, bqkv, Wo, bo,
           ln2_s, ln2_b, W1, b1, W2, b2, Wq_mem, bq_mem, Wr_mem, br_mem,
           out_s, out_b, K0, V0, S0, K1, V1, S1, K2, V2, S2):
    ids = input_ids.reshape(R).astype(jnp.int32)
    x = _addpos(_sc_gather(ids, tok_embed), pos_embed[:T])
    for l in range(NL):
        qkv = _qkv(x, ln1_s[l].reshape(1, D), ln1_b[l].reshape(1, D),
                   Wqkv[l], bqkv[l].reshape(1, 3 * D))
        o = _attn(qkv)
        x = _post(x, o, Wo[l], bo[l].reshape(1, D), ln2_s[l].reshape(1, D),
                  ln2_b[l].reshape(1, D), W1[l], b1[l].reshape(1, FF),
                  W2[l], b2[l].reshape(1, D))
    qm = _rowmm(x, Wq_mem, bq_mem.reshape(1, D))
    tvs, gs = [], []
    for Ki, Vi, Si in ((K0, V0, S0), (K1, V1, S1), (K2, V2, S2)):
        tv, ti = _topk(qm, Ki, Si.reshape(1, -1))
        tvs.append(tv)
        idxk = ti.T.reshape(TOPK * R)
        gs.append(_sc_gather(idxk, Vi).reshape(TOPK, R, D))
    xf = _final(x, gs[0], gs[1], gs[2], tvs[0], tvs[1], tvs[2], Wr_mem,
                br_mem.reshape(1, D), out_s.reshape(1, D), out_b.reshape(1, D))
    logits = _logits(xf, tok_embed)
    return logits.reshape(B, T, VOCAB)
